# Initial kernel scaffold; baseline (speedup 1.0000x reference)
import functools

import jax
import jax.numpy as jnp
from jax import lax
from jax.experimental import pallas as pl
from jax.experimental.pallas import tpu as pltpu
from jax.experimental.pallas import tpu_sc as plsc

N = 10000
E = 320000
D_IN = 128
H = 128
HH = H // 2
N_CLASSES = 64
K = 10

NUM_CORES = 2
NUM_TILES = 16
CHUNK = 128
EPT = 20096
NCH = EPT // CHUNK
E_PAD = EPT * NUM_TILES
RPT = N // NUM_TILES
RCH = 125
NRC = RPT // RCH


def _mlp_body(f_ref, w_ref, b_ref, o_ref):
    x = jnp.dot(f_ref[...], w_ref[...], preferred_element_type=jnp.float32)
    x = jnp.maximum(x + b_ref[...], 0.0)
    o_ref[0] = x[:, :HH]
    o_ref[1] = x[:, HH:]


def _proj_body(h_ref, w_ref, b_ref, o_ref):
    a = jnp.dot(h_ref[0], w_ref[:HH, :], preferred_element_type=jnp.float32)
    b = jnp.dot(h_ref[1], w_ref[HH:, :], preferred_element_type=jnp.float32)
    o_ref[...] = a + b + b_ref[...]


def _sc_body(x0, srcs, dsts, norms, temps, out,
             bufa, bufb, src_t, dst_t, norm_t, hid_t, rows_t, node_t,
             zero_t, temp_t):
    c = lax.axis_index("c")
    s = lax.axis_index("s")
    row0 = s * RPT

    pltpu.sync_copy(srcs.at[c, s], src_t)
    pltpu.sync_copy(dsts.at[c, s], dst_t)
    pltpu.sync_copy(norms.at[c, s], norm_t)
    pltpu.sync_copy(temps.at[c], temp_t)

    zf = jnp.zeros((16,), jnp.float32)

    def zrow(r, _):
        for q in range(4):
            zero_t[r, pl.ds(q * 16, 16)] = zf
        return 0

    lax.fori_loop(0, RCH, zrow, 0)

    t0 = [temp_t[0, pl.ds(q * 16, 16)] for q in range(4)]
    for j in range(NRC):
        base = row0 + j * RCH
        pltpu.sync_copy(x0.at[c, pl.ds(base, RCH)], node_t)
        pltpu.sync_copy(node_t, bufa.at[pl.ds(base, RCH)])

        def irow(r, _):
            for q in range(4):
                sl = pl.ds(q * 16, 16)
                hid_t[j * RCH + r, sl] = node_t[r, sl] * t0[q]
            return 0

        lax.fori_loop(0, RCH, irow, 0)
        pltpu.sync_copy(zero_t, bufb.at[pl.ds(base, RCH)])
    plsc.subcore_barrier()

    for k in range(K):
        src_buf, tgt_buf = (bufa, bufb) if k % 2 == 0 else (bufb, bufa)

        def echunk(j, _):
            pltpu.sync_copy(src_buf.at[src_t.at[j]], rows_t)

            def erow(i, _):
                nrm = norm_t[j, i]
                for q in range(4):
                    sl = pl.ds(q * 16, 16)
                    rows_t[i, sl] = rows_t[i, sl] * nrm
                return 0

            lax.fori_loop(0, CHUNK, erow, 0)
            pltpu.sync_copy(rows_t, tgt_buf.at[dst_t.at[j]], add=True)
            return 0

        lax.fori_loop(0, NCH, echunk, 0)
        plsc.subcore_barrier()

        tk = [temp_t[k + 1, pl.ds(q * 16, 16)] for q in range(4)]
        for j in range(NRC):
            base = row0 + j * RCH
            pltpu.sync_copy(tgt_buf.at[pl.ds(base, RCH)], node_t)

            def nrow(r, _):
                for q in range(4):
                    sl = pl.ds(q * 16, 16)
                    hid_t[j * RCH + r, sl] = (
                        hid_t[j * RCH + r, sl] + node_t[r, sl] * tk[q])
                return 0

            lax.fori_loop(0, RCH, nrow, 0)
            pltpu.sync_copy(zero_t, src_buf.at[pl.ds(base, RCH)])
        plsc.subcore_barrier()

    pltpu.sync_copy(hid_t, out.at[c, pl.ds(row0, RPT)])


_sc_prop = pl.kernel(
    _sc_body,
    out_type=jax.ShapeDtypeStruct((NUM_CORES, N, HH), jnp.float32),
    mesh=plsc.VectorSubcoreMesh(core_axis_name="c", subcore_axis_name="s"),
    scratch_types=[
        pltpu.VMEM_SHARED((N, HH), jnp.float32),
        pltpu.VMEM_SHARED((N, HH), jnp.float32),
        pltpu.VMEM((NCH, CHUNK), jnp.int32),
        pltpu.VMEM((NCH, CHUNK), jnp.int32),
        pltpu.VMEM((NCH, CHUNK), jnp.float32),
        pltpu.VMEM((RPT, HH), jnp.float32),
        pltpu.VMEM((CHUNK, HH), jnp.float32),
        pltpu.VMEM((RCH, HH), jnp.float32),
        pltpu.VMEM((RCH, HH), jnp.float32),
        pltpu.VMEM((K + 1, HH), jnp.float32),
    ],
)


def _pad_edges(idx, norm):
    pad = E_PAD - E
    src = jnp.concatenate([idx[0], jnp.zeros((pad,), jnp.int32)])
    dst = jnp.concatenate([idx[1], jnp.zeros((pad,), jnp.int32)])
    nrm = jnp.concatenate([norm, jnp.zeros((pad,), jnp.float32)])
    return (src.reshape(NUM_TILES, NCH, CHUNK),
            dst.reshape(NUM_TILES, NCH, CHUNK),
            nrm.reshape(NUM_TILES, NCH, CHUNK))


@jax.jit
def kernel(feature, edge_index, edge_index2, norm_A, norm_A_2,
           W1_1, b1_1, W1_2, b1_2, W2, b2, temp1, temp2):
    w1t = jnp.concatenate([W1_1, W1_2], axis=0).T
    b1 = jnp.concatenate([b1_1, b1_2])[None, :]

    blk = 1000
    x0 = pl.pallas_call(
        _mlp_body,
        grid=(N // blk,),
        in_specs=[
            pl.BlockSpec((blk, D_IN), lambda i: (i, 0)),
            pl.BlockSpec((D_IN, H), lambda i: (0, 0)),
            pl.BlockSpec((1, H), lambda i: (0, 0)),
        ],
        out_specs=pl.BlockSpec((NUM_CORES, blk, HH), lambda i: (0, i, 0)),
        out_shape=jax.ShapeDtypeStruct((NUM_CORES, N, HH), jnp.float32),
    )(feature, w1t, b1)

    s1, d1, n1 = _pad_edges(edge_index, norm_A)
    s2, d2, n2 = _pad_edges(edge_index2, norm_A_2)
    srcs = jnp.stack([s1, s2])
    dsts = jnp.stack([d1, d2])
    norms = jnp.stack([n1, n2])
    temps = jnp.stack([temp1.T, temp2.T])

    hidden = _sc_prop(x0, srcs, dsts, norms, temps)

    out = pl.pallas_call(
        _proj_body,
        grid=(N // blk,),
        in_specs=[
            pl.BlockSpec((NUM_CORES, blk, HH), lambda i: (0, i, 0)),
            pl.BlockSpec((H, N_CLASSES), lambda i: (0, 0)),
            pl.BlockSpec((1, N_CLASSES), lambda i: (0, 0)),
        ],
        out_specs=pl.BlockSpec((blk, N_CLASSES), lambda i: (i, 0)),
        out_shape=jax.ShapeDtypeStruct((N, N_CLASSES), jnp.float32),
    )(hidden, W2.T, b2[None, :])
    return out


# speedup vs baseline: 3.4947x; 3.4947x over previous
import jax
import jax.numpy as jnp
from jax import lax
from jax.experimental import pallas as pl
from jax.experimental.pallas import tpu as pltpu
from jax.experimental.pallas import tpu_sc as plsc

N = 10000
E = 320000
D_IN = 128
H = 128
HH = H // 2
N_CLASSES = 64
K = 10

NUM_CORES = 2
NUM_TILES = 16
CHUNK = 128
GRP = 32
NGRP = 5
NCH = GRP * NGRP
EPT = NCH * CHUNK
E_PAD = EPT * NUM_TILES
RPT = N // NUM_TILES
RCH = 125
NRC = RPT // RCH


def _mlp_body(f_ref, w_ref, b_ref, o_ref):
    x = jnp.dot(f_ref[...], w_ref[...], preferred_element_type=jnp.float32)
    x = jnp.maximum(x + b_ref[...], 0.0)
    o_ref[0] = x[:, :HH]
    o_ref[1] = x[:, HH:]


def _proj_body(x0_ref, h_ref, w_ref, b_ref, o_ref):
    acc = jnp.broadcast_to(b_ref[...], (x0_ref.shape[1], N_CLASSES))
    for c in range(NUM_CORES):
        acc = acc + jnp.dot(x0_ref[c], w_ref[c, 0],
                            preferred_element_type=jnp.float32)
        for k in range(K):
            acc = acc + jnp.dot(h_ref[c, k], w_ref[c, k + 1],
                                preferred_element_type=jnp.float32)
    o_ref[...] = acc


def _sc_body(x0, srcs, dsts, norms, out,
             bufa, bufb, src_st, dst_st, norm_st, rows_t, zero_t):
    c = lax.axis_index("c")
    s = lax.axis_index("s")
    row0 = s * RPT

    zf = jnp.zeros((16,), jnp.float32)

    def zrow(r, _):
        for q in range(4):
            zero_t[r, pl.ds(q * 16, 16)] = zf
        return 0

    lax.fori_loop(0, RCH, zrow, 0)

    pltpu.sync_copy(x0.at[c, pl.ds(row0, RPT)], bufa.at[pl.ds(row0, RPT)])
    for j in range(NRC):
        pltpu.sync_copy(zero_t, bufb.at[pl.ds(row0 + j * RCH, RCH)])
    plsc.subcore_barrier()

    for k in range(K):
        src_buf, tgt_buf = (bufa, bufb) if k % 2 == 0 else (bufb, bufa)

        def egrp(g, _):
            pltpu.sync_copy(srcs.at[c, s, pl.ds(g * GRP, GRP)], src_st)
            pltpu.sync_copy(dsts.at[c, s, pl.ds(g * GRP, GRP)], dst_st)
            pltpu.sync_copy(norms.at[c, s, pl.ds(g * GRP, GRP)], norm_st)

            def echunk(j, _):
                pltpu.sync_copy(src_buf.at[src_st.at[j]], rows_t)

                def egroup(gg, _):
                    nv = norm_st[j, pl.ds(gg * 16, 16)]
                    for l in range(16):
                        nrm = nv[l]
                        i = gg * 16 + l
                        for q in range(4):
                            sl = pl.ds(q * 16, 16)
                            rows_t[i, sl] = rows_t[i, sl] * nrm
                    return 0

                lax.fori_loop(0, CHUNK // 16, egroup, 0)
                pltpu.sync_copy(rows_t, tgt_buf.at[dst_st.at[j]], add=True)
                return 0

            lax.fori_loop(0, GRP, echunk, 0)
            return 0

        lax.fori_loop(0, NGRP, egrp, 0)
        plsc.subcore_barrier()

        pltpu.sync_copy(tgt_buf.at[pl.ds(row0, RPT)],
                        out.at[c, k, pl.ds(row0, RPT)])
        for j in range(NRC):
            pltpu.sync_copy(zero_t, src_buf.at[pl.ds(row0 + j * RCH, RCH)])
        plsc.subcore_barrier()


_sc_prop = pl.kernel(
    _sc_body,
    out_type=jax.ShapeDtypeStruct((NUM_CORES, K, N, HH), jnp.float32),
    mesh=plsc.VectorSubcoreMesh(core_axis_name="c", subcore_axis_name="s"),
    compiler_params=pltpu.CompilerParams(use_tc_tiling_on_sc=False),
    scratch_types=[
        pltpu.VMEM_SHARED((N, HH), jnp.float32),
        pltpu.VMEM_SHARED((N, HH), jnp.float32),
        pltpu.VMEM((GRP, CHUNK), jnp.int32),
        pltpu.VMEM((GRP, CHUNK), jnp.int32),
        pltpu.VMEM((GRP, CHUNK), jnp.float32),
        pltpu.VMEM((CHUNK, HH), jnp.float32),
        pltpu.VMEM((RCH, HH), jnp.float32),
    ],
)


def _pad_edges(idx, norm):
    pad = E_PAD - E
    src = jnp.concatenate([idx[0], jnp.zeros((pad,), jnp.int32)])
    dst = jnp.concatenate([idx[1], jnp.zeros((pad,), jnp.int32)])
    nrm = jnp.concatenate([norm, jnp.zeros((pad,), jnp.float32)])
    return (src.reshape(NUM_TILES, NCH, CHUNK),
            dst.reshape(NUM_TILES, NCH, CHUNK),
            nrm.reshape(NUM_TILES, NCH, CHUNK))


@jax.jit
def kernel(feature, edge_index, edge_index2, norm_A, norm_A_2,
           W1_1, b1_1, W1_2, b1_2, W2, b2, temp1, temp2):
    w1t = jnp.concatenate([W1_1, W1_2], axis=0).T
    b1 = jnp.concatenate([b1_1, b1_2])[None, :]

    blk = 1000
    x0 = pl.pallas_call(
        _mlp_body,
        grid=(N // blk,),
        in_specs=[
            pl.BlockSpec((blk, D_IN), lambda i: (i, 0)),
            pl.BlockSpec((D_IN, H), lambda i: (0, 0)),
            pl.BlockSpec((1, H), lambda i: (0, 0)),
        ],
        out_specs=pl.BlockSpec((NUM_CORES, blk, HH), lambda i: (0, i, 0)),
        out_shape=jax.ShapeDtypeStruct((NUM_CORES, N, HH), jnp.float32),
    )(feature, w1t, b1)

    s1, d1, n1 = _pad_edges(edge_index, norm_A)
    s2, d2, n2 = _pad_edges(edge_index2, norm_A_2)
    srcs = jnp.stack([s1, s2])
    dsts = jnp.stack([d1, d2])
    norms = jnp.stack([n1, n2])

    hops = _sc_prop(x0, srcs, dsts, norms)

    w2t = W2.T
    temps = jnp.stack([temp1.T, temp2.T])
    wsc = temps[:, :, :, None] * jnp.stack([w2t[:HH], w2t[HH:]])[:, None]

    out = pl.pallas_call(
        _proj_body,
        grid=(N // blk,),
        in_specs=[
            pl.BlockSpec((NUM_CORES, blk, HH), lambda i: (0, i, 0)),
            pl.BlockSpec((NUM_CORES, K, blk, HH), lambda i: (0, 0, i, 0)),
            pl.BlockSpec((NUM_CORES, K + 1, HH, N_CLASSES),
                         lambda i: (0, 0, 0, 0)),
            pl.BlockSpec((1, N_CLASSES), lambda i: (0, 0)),
        ],
        out_specs=pl.BlockSpec((blk, N_CLASSES), lambda i: (i, 0)),
        out_shape=jax.ShapeDtypeStruct((N, N_CLASSES), jnp.float32),
    )(x0, hops, wsc, b2[None, :])
    return out


# speedup vs baseline: 4.3539x; 1.2459x over previous
import jax
import jax.numpy as jnp
from jax import lax
from jax.experimental import pallas as pl
from jax.experimental.pallas import tpu as pltpu
from jax.experimental.pallas import tpu_sc as plsc

N = 10000
E = 320000
D_IN = 128
H = 128
HH = H // 2
N_CLASSES = 64
K = 10

NUM_CORES = 2
NUM_TILES = 16
CHUNK = 128
GRP = 16
NGRP = 10
NBUF = 4
NQ = GRP // NBUF
NCH = GRP * NGRP
EPT = NCH * CHUNK
E_PAD = EPT * NUM_TILES
RPT = N // NUM_TILES
RCH = 125
NRC = RPT // RCH


def _mlp_body(f_ref, w_ref, b_ref, o_ref):
    x = jnp.dot(f_ref[...], w_ref[...], preferred_element_type=jnp.float32)
    x = jnp.maximum(x + b_ref[...], 0.0)
    o_ref[0] = x[:, :HH]
    o_ref[1] = x[:, HH:]


def _proj_body(x0_ref, h_ref, w_ref, b_ref, o_ref):
    acc = jnp.broadcast_to(b_ref[...], (x0_ref.shape[1], N_CLASSES))
    for c in range(NUM_CORES):
        acc = acc + jnp.dot(x0_ref[c], w_ref[c, 0],
                            preferred_element_type=jnp.float32)
        for k in range(K):
            acc = acc + jnp.dot(h_ref[c, k], w_ref[c, k + 1],
                                preferred_element_type=jnp.float32)
    o_ref[...] = acc


def _scale_rows(rows, norm_st, j):
    def egroup(gg, _):
        nv = norm_st[j, pl.ds(gg * 16, 16)]
        for l in range(16):
            nrm = nv[l]
            i = gg * 16 + l
            for q in range(4):
                sl = pl.ds(q * 16, 16)
                rows[i, sl] = rows[i, sl] * nrm
        return 0

    lax.fori_loop(0, CHUNK // 16, egroup, 0)


def _sc_body(x0, srcs, dsts, norms, out,
             bufa, bufb, src_st, dst_st, norm_st,
             rows0, rows1, rows2, rows3, zero_t, gsem, ssem):
    c = lax.axis_index("c")
    s = lax.axis_index("s")
    row0 = s * RPT
    rows = [rows0, rows1, rows2, rows3]

    zf = jnp.zeros((16,), jnp.float32)

    def zrow(r, _):
        for q in range(4):
            zero_t[r, pl.ds(q * 16, 16)] = zf
        return 0

    lax.fori_loop(0, RCH, zrow, 0)

    pltpu.sync_copy(x0.at[c, pl.ds(row0, RPT)], bufa.at[pl.ds(row0, RPT)])
    for j in range(NRC):
        pltpu.sync_copy(zero_t, bufb.at[pl.ds(row0 + j * RCH, RCH)])
    plsc.subcore_barrier()

    def hop(k, src_buf, tgt_buf):
        def egrp(g, _):
            pltpu.sync_copy(srcs.at[c, s, pl.ds(g * GRP, GRP)], src_st)
            pltpu.sync_copy(dsts.at[c, s, pl.ds(g * GRP, GRP)], dst_st)
            pltpu.sync_copy(norms.at[c, s, pl.ds(g * GRP, GRP)], norm_st)

            def equad(q, _):
                qq = g * NQ + q
                gds = []
                for t in range(NBUF):
                    j = q * NBUF + t

                    @pl.when(qq > 0)
                    def _():
                        pltpu.make_async_copy(
                            rows[t], tgt_buf.at[dst_st.at[j]],
                            ssem.at[t]).wait()

                    gds.append(pltpu.async_copy(
                        src_buf.at[src_st.at[j]], rows[t], gsem.at[t]))
                for t in range(NBUF):
                    j = q * NBUF + t
                    gds[t].wait()
                    _scale_rows(rows[t], norm_st, j)
                    pltpu.async_copy(rows[t], tgt_buf.at[dst_st.at[j]],
                                     ssem.at[t], add=True)
                return 0

            lax.fori_loop(0, NQ, equad, 0)
            return 0

        lax.fori_loop(0, NGRP, egrp, 0)
        for t in range(NBUF):
            pltpu.make_async_copy(rows[t], tgt_buf.at[dst_st.at[t]],
                                  ssem.at[t]).wait()
        plsc.subcore_barrier()

        pltpu.sync_copy(tgt_buf.at[pl.ds(row0, RPT)],
                        out.at[c, k, pl.ds(row0, RPT)])
        for j in range(NRC):
            pltpu.sync_copy(zero_t, src_buf.at[pl.ds(row0 + j * RCH, RCH)])
        plsc.subcore_barrier()

    def hop_pair(kk, _):
        hop(2 * kk, bufa, bufb)
        hop(2 * kk + 1, bufb, bufa)
        return 0

    lax.fori_loop(0, K // 2, hop_pair, 0)


_sc_prop = pl.kernel(
    _sc_body,
    out_type=jax.ShapeDtypeStruct((NUM_CORES, K, N, HH), jnp.float32),
    mesh=plsc.VectorSubcoreMesh(core_axis_name="c", subcore_axis_name="s"),
    compiler_params=pltpu.CompilerParams(use_tc_tiling_on_sc=False),
    scratch_types=[
        pltpu.VMEM_SHARED((N, HH), jnp.float32),
        pltpu.VMEM_SHARED((N, HH), jnp.float32),
        pltpu.VMEM((GRP, CHUNK), jnp.int32),
        pltpu.VMEM((GRP, CHUNK), jnp.int32),
        pltpu.VMEM((GRP, CHUNK), jnp.float32),
        pltpu.VMEM((CHUNK, HH), jnp.float32),
        pltpu.VMEM((CHUNK, HH), jnp.float32),
        pltpu.VMEM((CHUNK, HH), jnp.float32),
        pltpu.VMEM((CHUNK, HH), jnp.float32),
        pltpu.VMEM((RCH, HH), jnp.float32),
        pltpu.SemaphoreType.DMA((NBUF,)),
        pltpu.SemaphoreType.DMA((NBUF,)),
    ],
)


def _pad_edges(idx, norm):
    pad = E_PAD - E
    src = jnp.concatenate([idx[0], jnp.zeros((pad,), jnp.int32)])
    dst = jnp.concatenate([idx[1], jnp.zeros((pad,), jnp.int32)])
    nrm = jnp.concatenate([norm, jnp.zeros((pad,), jnp.float32)])
    return (src.reshape(NUM_TILES, NCH, CHUNK),
            dst.reshape(NUM_TILES, NCH, CHUNK),
            nrm.reshape(NUM_TILES, NCH, CHUNK))


@jax.jit
def kernel(feature, edge_index, edge_index2, norm_A, norm_A_2,
           W1_1, b1_1, W1_2, b1_2, W2, b2, temp1, temp2):
    w1t = jnp.concatenate([W1_1, W1_2], axis=0).T
    b1 = jnp.concatenate([b1_1, b1_2])[None, :]

    blk = 1000
    x0 = pl.pallas_call(
        _mlp_body,
        grid=(N // blk,),
        in_specs=[
            pl.BlockSpec((blk, D_IN), lambda i: (i, 0)),
            pl.BlockSpec((D_IN, H), lambda i: (0, 0)),
            pl.BlockSpec((1, H), lambda i: (0, 0)),
        ],
        out_specs=pl.BlockSpec((NUM_CORES, blk, HH), lambda i: (0, i, 0)),
        out_shape=jax.ShapeDtypeStruct((NUM_CORES, N, HH), jnp.float32),
    )(feature, w1t, b1)

    s1, d1, n1 = _pad_edges(edge_index, norm_A)
    s2, d2, n2 = _pad_edges(edge_index2, norm_A_2)
    srcs = jnp.stack([s1, s2])
    dsts = jnp.stack([d1, d2])
    norms = jnp.stack([n1, n2])

    hops = _sc_prop(x0, srcs, dsts, norms)

    w2t = W2.T
    temps = jnp.stack([temp1.T, temp2.T])
    wsc = temps[:, :, :, None] * jnp.stack([w2t[:HH], w2t[HH:]])[:, None]

    out = pl.pallas_call(
        _proj_body,
        grid=(N // blk,),
        in_specs=[
            pl.BlockSpec((NUM_CORES, blk, HH), lambda i: (0, i, 0)),
            pl.BlockSpec((NUM_CORES, K, blk, HH), lambda i: (0, 0, i, 0)),
            pl.BlockSpec((NUM_CORES, K + 1, HH, N_CLASSES),
                         lambda i: (0, 0, 0, 0)),
            pl.BlockSpec((1, N_CLASSES), lambda i: (0, 0)),
        ],
        out_specs=pl.BlockSpec((blk, N_CLASSES), lambda i: (i, 0)),
        out_shape=jax.ShapeDtypeStruct((N, N_CLASSES), jnp.float32),
    )(x0, hops, wsc, b2[None, :])
    return out


# speedup vs baseline: 4.3556x; 1.0004x over previous
import jax
import jax.numpy as jnp
from jax import lax
from jax.experimental import pallas as pl
from jax.experimental.pallas import tpu as pltpu
from jax.experimental.pallas import tpu_sc as plsc

N = 10000
E = 320000
D_IN = 128
H = 128
HH = H // 2
N_CLASSES = 64
K = 10

NUM_CORES = 2
NUM_TILES = 16
CHUNK = 128
GRP = 16
NGRP = 10
NBUF = 4
NQ = GRP // NBUF
NCH = GRP * NGRP
EPT = NCH * CHUNK
E_PAD = EPT * NUM_TILES
RPT = N // NUM_TILES
RCH = 125
NRC = RPT // RCH


def _mlp_body(f_ref, w_ref, b_ref, o_ref):
    x = jnp.dot(f_ref[...], w_ref[...], preferred_element_type=jnp.float32)
    x = jnp.maximum(x + b_ref[...], 0.0)
    o_ref[0] = x[:, :HH]
    o_ref[1] = x[:, HH:]


def _proj_body(x0_ref, h_ref, w_ref, b_ref, o_ref):
    acc = jnp.broadcast_to(b_ref[...], (x0_ref.shape[1], N_CLASSES))
    for c in range(NUM_CORES):
        acc = acc + jnp.dot(x0_ref[c], w_ref[c, 0],
                            preferred_element_type=jnp.float32)
        for k in range(K):
            acc = acc + jnp.dot(h_ref[c, k], w_ref[c, k + 1],
                                preferred_element_type=jnp.float32)
    o_ref[...] = acc


def _scale_rows(rows, norm_st, j):
    dn = lax.GatherDimensionNumbers(
        offset_dims=(), collapsed_slice_dims=(0,), start_index_map=(0,))

    def egroup(gg, _):
        nv = norm_st[j, pl.ds(gg * 16, 16)]
        for l in range(16):
            nsp = lax.gather(
                nv, jnp.full((16, 1), l, jnp.int32), dn, slice_sizes=(1,),
                mode=lax.GatherScatterMode.PROMISE_IN_BOUNDS)
            i = gg * 16 + l
            for q in range(4):
                sl = pl.ds(q * 16, 16)
                rows[i, sl] = rows[i, sl] * nsp
        return 0

    lax.fori_loop(0, CHUNK // 16, egroup, 0)


def _sc_body(x0, srcs, dsts, norms, out,
             bufa, bufb, src_st, dst_st, norm_st,
             rows0, rows1, rows2, rows3, zero_t, gsem, ssem):
    c = lax.axis_index("c")
    s = lax.axis_index("s")
    row0 = s * RPT
    rows = [rows0, rows1, rows2, rows3]

    zf = jnp.zeros((16,), jnp.float32)

    def zrow(r, _):
        for q in range(4):
            zero_t[r, pl.ds(q * 16, 16)] = zf
        return 0

    lax.fori_loop(0, RCH, zrow, 0)

    pltpu.sync_copy(x0.at[c, pl.ds(row0, RPT)], bufa.at[pl.ds(row0, RPT)])
    for j in range(NRC):
        pltpu.sync_copy(zero_t, bufb.at[pl.ds(row0 + j * RCH, RCH)])
    plsc.subcore_barrier()

    def hop(k, src_buf, tgt_buf):
        def egrp(g, _):
            pltpu.sync_copy(srcs.at[c, s, pl.ds(g * GRP, GRP)], src_st)
            pltpu.sync_copy(dsts.at[c, s, pl.ds(g * GRP, GRP)], dst_st)
            pltpu.sync_copy(norms.at[c, s, pl.ds(g * GRP, GRP)], norm_st)

            def equad(q, _):
                qq = g * NQ + q
                gds = []
                for t in range(NBUF):
                    j = q * NBUF + t

                    @pl.when(qq > 0)
                    def _():
                        pltpu.make_async_copy(
                            rows[t], tgt_buf.at[dst_st.at[j]],
                            ssem.at[t]).wait()

                    gds.append(pltpu.async_copy(
                        src_buf.at[src_st.at[j]], rows[t], gsem.at[t]))
                for t in range(NBUF):
                    j = q * NBUF + t
                    gds[t].wait()
                    _scale_rows(rows[t], norm_st, j)
                    pltpu.async_copy(rows[t], tgt_buf.at[dst_st.at[j]],
                                     ssem.at[t], add=True)
                return 0

            lax.fori_loop(0, NQ, equad, 0)
            return 0

        lax.fori_loop(0, NGRP, egrp, 0)
        for t in range(NBUF):
            pltpu.make_async_copy(rows[t], tgt_buf.at[dst_st.at[t]],
                                  ssem.at[t]).wait()
        plsc.subcore_barrier()

        pltpu.sync_copy(tgt_buf.at[pl.ds(row0, RPT)],
                        out.at[c, k, pl.ds(row0, RPT)])
        for j in range(NRC):
            pltpu.sync_copy(zero_t, src_buf.at[pl.ds(row0 + j * RCH, RCH)])
        plsc.subcore_barrier()

    def hop_pair(kk, _):
        hop(2 * kk, bufa, bufb)
        hop(2 * kk + 1, bufb, bufa)
        return 0

    lax.fori_loop(0, K // 2, hop_pair, 0)


_sc_prop = pl.kernel(
    _sc_body,
    out_type=jax.ShapeDtypeStruct((NUM_CORES, K, N, HH), jnp.float32),
    mesh=plsc.VectorSubcoreMesh(core_axis_name="c", subcore_axis_name="s"),
    compiler_params=pltpu.CompilerParams(use_tc_tiling_on_sc=False),
    scratch_types=[
        pltpu.VMEM_SHARED((N, HH), jnp.float32),
        pltpu.VMEM_SHARED((N, HH), jnp.float32),
        pltpu.VMEM((GRP, CHUNK), jnp.int32),
        pltpu.VMEM((GRP, CHUNK), jnp.int32),
        pltpu.VMEM((GRP, CHUNK), jnp.float32),
        pltpu.VMEM((CHUNK, HH), jnp.float32),
        pltpu.VMEM((CHUNK, HH), jnp.float32),
        pltpu.VMEM((CHUNK, HH), jnp.float32),
        pltpu.VMEM((CHUNK, HH), jnp.float32),
        pltpu.VMEM((RCH, HH), jnp.float32),
        pltpu.SemaphoreType.DMA((NBUF,)),
        pltpu.SemaphoreType.DMA((NBUF,)),
    ],
)


def _pad_edges(idx, norm):
    pad = E_PAD - E
    src = jnp.concatenate([idx[0], jnp.zeros((pad,), jnp.int32)])
    dst = jnp.concatenate([idx[1], jnp.zeros((pad,), jnp.int32)])
    nrm = jnp.concatenate([norm, jnp.zeros((pad,), jnp.float32)])
    return (src.reshape(NUM_TILES, NCH, CHUNK),
            dst.reshape(NUM_TILES, NCH, CHUNK),
            nrm.reshape(NUM_TILES, NCH, CHUNK))


@jax.jit
def kernel(feature, edge_index, edge_index2, norm_A, norm_A_2,
           W1_1, b1_1, W1_2, b1_2, W2, b2, temp1, temp2):
    w1t = jnp.concatenate([W1_1, W1_2], axis=0).T
    b1 = jnp.concatenate([b1_1, b1_2])[None, :]

    blk = 1000
    x0 = pl.pallas_call(
        _mlp_body,
        grid=(N // blk,),
        in_specs=[
            pl.BlockSpec((blk, D_IN), lambda i: (i, 0)),
            pl.BlockSpec((D_IN, H), lambda i: (0, 0)),
            pl.BlockSpec((1, H), lambda i: (0, 0)),
        ],
        out_specs=pl.BlockSpec((NUM_CORES, blk, HH), lambda i: (0, i, 0)),
        out_shape=jax.ShapeDtypeStruct((NUM_CORES, N, HH), jnp.float32),
    )(feature, w1t, b1)

    s1, d1, n1 = _pad_edges(edge_index, norm_A)
    s2, d2, n2 = _pad_edges(edge_index2, norm_A_2)
    srcs = jnp.stack([s1, s2])
    dsts = jnp.stack([d1, d2])
    norms = jnp.stack([n1, n2])

    hops = _sc_prop(x0, srcs, dsts, norms)

    w2t = W2.T
    temps = jnp.stack([temp1.T, temp2.T])
    wsc = temps[:, :, :, None] * jnp.stack([w2t[:HH], w2t[HH:]])[:, None]

    out = pl.pallas_call(
        _proj_body,
        grid=(N // blk,),
        in_specs=[
            pl.BlockSpec((NUM_CORES, blk, HH), lambda i: (0, i, 0)),
            pl.BlockSpec((NUM_CORES, K, blk, HH), lambda i: (0, 0, i, 0)),
            pl.BlockSpec((NUM_CORES, K + 1, HH, N_CLASSES),
                         lambda i: (0, 0, 0, 0)),
            pl.BlockSpec((1, N_CLASSES), lambda i: (0, 0)),
        ],
        out_specs=pl.BlockSpec((blk, N_CLASSES), lambda i: (i, 0)),
        out_shape=jax.ShapeDtypeStruct((N, N_CLASSES), jnp.float32),
    )(x0, hops, wsc, b2[None, :])
    return out


# speedup vs baseline: 9.4065x; 2.1596x over previous
import jax
import jax.numpy as jnp
from jax import lax
from jax.experimental import pallas as pl
from jax.experimental.pallas import tpu as pltpu
from jax.experimental.pallas import tpu_sc as plsc

N = 10000
E = 320000
D_IN = 128
H = 128
HH = H // 2
N_CLASSES = 64
K = 10

NUM_CORES = 2
NUM_TILES = 16
CHUNK = 128
GRP = 16
NGRP = 10
NBUF = 4
NQ = GRP // NBUF
NCH = GRP * NGRP
EPT = NCH * CHUNK
E_PAD = EPT * NUM_TILES
RPT = N // NUM_TILES
RCH = 125
NRC = RPT // RCH


def _mlp_body(f_ref, w_ref, b_ref, o_ref):
    x = jnp.dot(f_ref[...], w_ref[...], preferred_element_type=jnp.float32)
    x = jnp.maximum(x + b_ref[...], 0.0)
    o_ref[0] = x[:, :HH]
    o_ref[1] = x[:, HH:]


def _proj_body(x0_ref, h_ref, w_ref, b_ref, o_ref):
    acc = jnp.broadcast_to(b_ref[...], (x0_ref.shape[1], N_CLASSES))
    for c in range(NUM_CORES):
        acc = acc + jnp.dot(x0_ref[c], w_ref[c, 0],
                            preferred_element_type=jnp.float32)
        for k in range(K):
            acc = acc + jnp.dot(h_ref[c, k], w_ref[c, k + 1],
                                preferred_element_type=jnp.float32)
    o_ref[...] = acc


def _scale_copy(gb, sb, norm_st, j):
    dn = lax.GatherDimensionNumbers(
        offset_dims=(), collapsed_slice_dims=(0,), start_index_map=(0,))

    def egroup(gg, _):
        nv = norm_st[j, pl.ds(gg * 16, 16)]
        for l in range(16):
            nsp = lax.gather(
                nv, jnp.full((16, 1), l, jnp.int32), dn, slice_sizes=(1,),
                mode=lax.GatherScatterMode.PROMISE_IN_BOUNDS)
            i = gg * 16 + l
            for q in range(4):
                sl = pl.ds(q * 16, 16)
                sb[i, sl] = gb[i, sl] * nsp
        return 0

    lax.fori_loop(0, CHUNK // 16, egroup, 0)


def _sc_body(x0, srcs, dsts, norms, out,
             bufa, bufb, src_st, dst_st, norm_st,
             rows0, rows1, rows2, rows3, zero_t, gsem, ssem):
    c = lax.axis_index("c")
    s = lax.axis_index("s")
    row0 = s * RPT
    rows = [rows0, rows1, rows2, rows3]

    zf = jnp.zeros((16,), jnp.float32)

    def zrow(r, _):
        for q in range(4):
            zero_t[r, pl.ds(q * 16, 16)] = zf
        return 0

    lax.fori_loop(0, RCH, zrow, 0)

    pltpu.sync_copy(x0.at[c, pl.ds(row0, RPT)], bufa.at[pl.ds(row0, RPT)])
    for j in range(NRC):
        pltpu.sync_copy(zero_t, bufb.at[pl.ds(row0 + j * RCH, RCH)])
    plsc.subcore_barrier()

    def hop(k, src_buf, tgt_buf):
        gb = [rows[0], rows[1]]
        sb = [rows[2], rows[3]]

        def egrp(g, _):
            pltpu.sync_copy(srcs.at[c, s, pl.ds(g * GRP, GRP)], src_st)
            pltpu.sync_copy(dsts.at[c, s, pl.ds(g * GRP, GRP)], dst_st)
            pltpu.sync_copy(norms.at[c, s, pl.ds(g * GRP, GRP)], norm_st)
            for t in range(2):
                pltpu.async_copy(src_buf.at[src_st.at[t]], gb[t], gsem.at[t])

            def epair(m, _):
                for t in range(2):
                    j = 2 * m + t
                    pltpu.make_async_copy(src_buf.at[src_st.at[j]],
                                          gb[t], gsem.at[t]).wait()

                    @pl.when(g * GRP + j >= 2)
                    def _():
                        pltpu.make_async_copy(
                            sb[t], tgt_buf.at[dst_st.at[j]],
                            ssem.at[t]).wait()

                    _scale_copy(gb[t], sb[t], norm_st, j)
                    pltpu.async_copy(sb[t], tgt_buf.at[dst_st.at[j]],
                                     ssem.at[t], add=True)

                    @pl.when(j + 2 < GRP)
                    def _():
                        pltpu.async_copy(src_buf.at[src_st.at[j + 2]],
                                         gb[t], gsem.at[t])
                return 0

            lax.fori_loop(0, GRP // 2, epair, 0)
            return 0

        lax.fori_loop(0, NGRP, egrp, 0)
        for t in range(2):
            pltpu.make_async_copy(sb[t], tgt_buf.at[dst_st.at[t]],
                                  ssem.at[t]).wait()
        plsc.subcore_barrier()

        pltpu.sync_copy(tgt_buf.at[pl.ds(row0, RPT)],
                        out.at[c, k, pl.ds(row0, RPT)])
        for j in range(NRC):
            pltpu.sync_copy(zero_t, src_buf.at[pl.ds(row0 + j * RCH, RCH)])
        plsc.subcore_barrier()

    def hop_pair(kk, _):
        hop(2 * kk, bufa, bufb)
        hop(2 * kk + 1, bufb, bufa)
        return 0

    lax.fori_loop(0, K // 2, hop_pair, 0)


_sc_prop = pl.kernel(
    _sc_body,
    out_type=jax.ShapeDtypeStruct((NUM_CORES, K, N, HH), jnp.float32),
    mesh=plsc.VectorSubcoreMesh(core_axis_name="c", subcore_axis_name="s"),
    compiler_params=pltpu.CompilerParams(use_tc_tiling_on_sc=False),
    scratch_types=[
        pltpu.VMEM_SHARED((N, HH), jnp.float32),
        pltpu.VMEM_SHARED((N, HH), jnp.float32),
        pltpu.VMEM((GRP, CHUNK), jnp.int32),
        pltpu.VMEM((GRP, CHUNK), jnp.int32),
        pltpu.VMEM((GRP, CHUNK), jnp.float32),
        pltpu.VMEM((CHUNK, HH), jnp.float32),
        pltpu.VMEM((CHUNK, HH), jnp.float32),
        pltpu.VMEM((CHUNK, HH), jnp.float32),
        pltpu.VMEM((CHUNK, HH), jnp.float32),
        pltpu.VMEM((RCH, HH), jnp.float32),
        pltpu.SemaphoreType.DMA((NBUF,)),
        pltpu.SemaphoreType.DMA((NBUF,)),
    ],
)


def _pad_edges(idx, norm):
    pad = E_PAD - E
    src = jnp.concatenate([idx[0], jnp.zeros((pad,), jnp.int32)])
    dst = jnp.concatenate([idx[1], jnp.zeros((pad,), jnp.int32)])
    nrm = jnp.concatenate([norm, jnp.zeros((pad,), jnp.float32)])
    return (src.reshape(NUM_TILES, NCH, CHUNK),
            dst.reshape(NUM_TILES, NCH, CHUNK),
            nrm.reshape(NUM_TILES, NCH, CHUNK))


@jax.jit
def kernel(feature, edge_index, edge_index2, norm_A, norm_A_2,
           W1_1, b1_1, W1_2, b1_2, W2, b2, temp1, temp2):
    w1t = jnp.concatenate([W1_1, W1_2], axis=0).T
    b1 = jnp.concatenate([b1_1, b1_2])[None, :]

    blk = 1000
    x0 = pl.pallas_call(
        _mlp_body,
        grid=(N // blk,),
        in_specs=[
            pl.BlockSpec((blk, D_IN), lambda i: (i, 0)),
            pl.BlockSpec((D_IN, H), lambda i: (0, 0)),
            pl.BlockSpec((1, H), lambda i: (0, 0)),
        ],
        out_specs=pl.BlockSpec((NUM_CORES, blk, HH), lambda i: (0, i, 0)),
        out_shape=jax.ShapeDtypeStruct((NUM_CORES, N, HH), jnp.float32),
    )(feature, w1t, b1)

    s1, d1, n1 = _pad_edges(edge_index, norm_A)
    s2, d2, n2 = _pad_edges(edge_index2, norm_A_2)
    srcs = jnp.stack([s1, s2])
    dsts = jnp.stack([d1, d2])
    norms = jnp.stack([n1, n2])

    hops = _sc_prop(x0, srcs, dsts, norms)

    w2t = W2.T
    temps = jnp.stack([temp1.T, temp2.T])
    wsc = temps[:, :, :, None] * jnp.stack([w2t[:HH], w2t[HH:]])[:, None]

    out = pl.pallas_call(
        _proj_body,
        grid=(N // blk,),
        in_specs=[
            pl.BlockSpec((NUM_CORES, blk, HH), lambda i: (0, i, 0)),
            pl.BlockSpec((NUM_CORES, K, blk, HH), lambda i: (0, 0, i, 0)),
            pl.BlockSpec((NUM_CORES, K + 1, HH, N_CLASSES),
                         lambda i: (0, 0, 0, 0)),
            pl.BlockSpec((1, N_CLASSES), lambda i: (0, 0)),
        ],
        out_specs=pl.BlockSpec((blk, N_CLASSES), lambda i: (i, 0)),
        out_shape=jax.ShapeDtypeStruct((N, N_CLASSES), jnp.float32),
    )(x0, hops, wsc, b2[None, :])
    return out


# speedup vs baseline: 10.8020x; 1.1484x over previous
import jax
import jax.numpy as jnp
from jax import lax
from jax.experimental import pallas as pl
from jax.experimental.pallas import tpu as pltpu
from jax.experimental.pallas import tpu_sc as plsc

N = 10000
E = 320000
D_IN = 128
H = 128
HH = H // 2
N_CLASSES = 64
K = 10

NUM_CORES = 2
NUM_TILES = 16
CHUNK = 128
GRP = 16
NGRP = 10
NBUF = 4
NQ = GRP // NBUF
NCH = GRP * NGRP
EPT = NCH * CHUNK
E_PAD = EPT * NUM_TILES
RPT = N // NUM_TILES
RCH = 125
NRC = RPT // RCH


def _mlp_body(f_ref, w_ref, b_ref, o_ref):
    x = jnp.dot(f_ref[...], w_ref[...], preferred_element_type=jnp.float32)
    x = jnp.maximum(x + b_ref[...], 0.0)
    o_ref[0] = x[:, :HH]
    o_ref[1] = x[:, HH:]


def _proj_body(x0_ref, h_ref, w_ref, b_ref, o_ref):
    acc = jnp.broadcast_to(b_ref[...], (x0_ref.shape[1], N_CLASSES))
    for c in range(NUM_CORES):
        acc = acc + jnp.dot(x0_ref[c], w_ref[c, 0],
                            preferred_element_type=jnp.float32)
        for k in range(K):
            acc = acc + jnp.dot(h_ref[c, k], w_ref[c, k + 1],
                                preferred_element_type=jnp.float32)
    o_ref[...] = acc


def _scale_copy(gb, sb, norm_st, j):
    dn = lax.GatherDimensionNumbers(
        offset_dims=(), collapsed_slice_dims=(0,), start_index_map=(0,))

    def egroup(gg, _):
        nv = norm_st[j, pl.ds(gg * 16, 16)]
        for l in range(16):
            nsp = lax.gather(
                nv, jnp.full((16, 1), l, jnp.int32), dn, slice_sizes=(1,),
                mode=lax.GatherScatterMode.PROMISE_IN_BOUNDS)
            i = gg * 16 + l
            for q in range(4):
                sl = pl.ds(q * 16, 16)
                sb[i, sl] = gb[i, sl] * nsp
        return 0

    lax.fori_loop(0, CHUNK // 16, egroup, 0)


def _sc_body(x0, srcs, dsts, norms, out,
             bufa, bufb, src_st, dst_st, norm_st,
             rows0, rows1, rows2, rows3, gsem, ssem, stgsem, outsem):
    c = lax.axis_index("c")
    s = lax.axis_index("s")
    row0 = s * RPT
    gb = [rows0, rows1]
    sb = [rows2, rows3]
    zf = jnp.zeros((16,), jnp.float32)

    def zero_sb0():
        def zrow(r, _):
            for q in range(4):
                sb[0][r, pl.ds(q * 16, 16)] = zf
            return 0

        lax.fori_loop(0, CHUNK, zrow, 0)

    def zero_slice(buf):
        for i in range(4):
            pltpu.sync_copy(sb[0], buf.at[pl.ds(row0 + i * CHUNK, CHUNK)])
        pltpu.sync_copy(sb[0].at[pl.ds(0, RPT - 4 * CHUNK)],
                        buf.at[pl.ds(row0 + 4 * CHUNK, RPT - 4 * CHUNK)])

    zero_sb0()
    pltpu.sync_copy(x0.at[c, pl.ds(row0, RPT)], bufa.at[pl.ds(row0, RPT)])
    zero_slice(bufb)
    plsc.subcore_barrier()

    def stage_issue(g, p):
        pltpu.async_copy(srcs.at[c, s, pl.ds(g * GRP, GRP)], src_st.at[p],
                         stgsem.at[p])
        pltpu.async_copy(dsts.at[c, s, pl.ds(g * GRP, GRP)], dst_st.at[p],
                         stgsem.at[p])
        pltpu.async_copy(norms.at[c, s, pl.ds(g * GRP, GRP)], norm_st.at[p],
                         stgsem.at[p])

    def stage_wait(g, p):
        pltpu.make_async_copy(srcs.at[c, s, pl.ds(g * GRP, GRP)],
                              src_st.at[p], stgsem.at[p]).wait()
        pltpu.make_async_copy(dsts.at[c, s, pl.ds(g * GRP, GRP)],
                              dst_st.at[p], stgsem.at[p]).wait()
        pltpu.make_async_copy(norms.at[c, s, pl.ds(g * GRP, GRP)],
                              norm_st.at[p], stgsem.at[p]).wait()

    def hop(k, src_buf, tgt_buf):
        stage_issue(0, 0)
        stage_wait(0, 0)
        for t in range(2):
            pltpu.async_copy(src_buf.at[src_st.at[0, t]], gb[t], gsem.at[t])

        def egrp(g, _):
            p = lax.rem(g, 2)

            @pl.when(g + 1 < NGRP)
            def _():
                stage_issue(g + 1, 1 - p)

            def epair(m, _):
                for t in range(2):
                    j = 2 * m + t
                    pltpu.make_async_copy(src_buf.at[src_st.at[p, j]],
                                          gb[t], gsem.at[t]).wait()

                    @pl.when(g * GRP + j >= 2)
                    def _():
                        pltpu.make_async_copy(
                            sb[t], tgt_buf.at[dst_st.at[p, j]],
                            ssem.at[t]).wait()

                    _scale_copy(gb[t], sb[t], norm_st.at[p], j)
                    pltpu.async_copy(sb[t], tgt_buf.at[dst_st.at[p, j]],
                                     ssem.at[t], add=True)

                    @pl.when(j + 2 < GRP)
                    def _():
                        pltpu.async_copy(src_buf.at[src_st.at[p, j + 2]],
                                         gb[t], gsem.at[t])
                return 0

            lax.fori_loop(0, GRP // 2, epair, 0)

            @pl.when(g + 1 < NGRP)
            def _():
                stage_wait(g + 1, 1 - p)
                for t in range(2):
                    pltpu.async_copy(src_buf.at[src_st.at[1 - p, t]],
                                     gb[t], gsem.at[t])
            return 0

        lax.fori_loop(0, NGRP, egrp, 0)
        for t in range(2):
            pltpu.make_async_copy(sb[t], tgt_buf.at[dst_st.at[0, t]],
                                  ssem.at[t]).wait()
        plsc.subcore_barrier()

        pltpu.async_copy(tgt_buf.at[pl.ds(row0, RPT)],
                         out.at[c, k, pl.ds(row0, RPT)], outsem)

        @pl.when(k > 0)
        def _():
            pltpu.make_async_copy(src_buf.at[pl.ds(row0, RPT)],
                                  out.at[c, k - 1, pl.ds(row0, RPT)],
                                  outsem).wait()

        zero_sb0()
        zero_slice(src_buf)
        plsc.subcore_barrier()

    def hop_pair(kk, _):
        hop(2 * kk, bufa, bufb)
        hop(2 * kk + 1, bufb, bufa)
        return 0

    lax.fori_loop(0, K // 2, hop_pair, 0)
    pltpu.make_async_copy(bufa.at[pl.ds(row0, RPT)],
                          out.at[c, K - 1, pl.ds(row0, RPT)], outsem).wait()


_sc_prop = pl.kernel(
    _sc_body,
    out_type=jax.ShapeDtypeStruct((NUM_CORES, K, N, HH), jnp.float32),
    mesh=plsc.VectorSubcoreMesh(core_axis_name="c", subcore_axis_name="s"),
    compiler_params=pltpu.CompilerParams(use_tc_tiling_on_sc=False),
    scratch_types=[
        pltpu.VMEM_SHARED((N, HH), jnp.float32),
        pltpu.VMEM_SHARED((N, HH), jnp.float32),
        pltpu.VMEM((2, GRP, CHUNK), jnp.int32),
        pltpu.VMEM((2, GRP, CHUNK), jnp.int32),
        pltpu.VMEM((2, GRP, CHUNK), jnp.float32),
        pltpu.VMEM((CHUNK, HH), jnp.float32),
        pltpu.VMEM((CHUNK, HH), jnp.float32),
        pltpu.VMEM((CHUNK, HH), jnp.float32),
        pltpu.VMEM((CHUNK, HH), jnp.float32),
        pltpu.SemaphoreType.DMA((2,)),
        pltpu.SemaphoreType.DMA((2,)),
        pltpu.SemaphoreType.DMA((2,)),
        pltpu.SemaphoreType.DMA,
    ],
)


def _pad_edges(idx, norm):
    pad = E_PAD - E
    src = jnp.concatenate([idx[0], jnp.zeros((pad,), jnp.int32)])
    dst = jnp.concatenate([idx[1], jnp.zeros((pad,), jnp.int32)])
    nrm = jnp.concatenate([norm, jnp.zeros((pad,), jnp.float32)])
    return (src.reshape(NUM_TILES, NCH, CHUNK),
            dst.reshape(NUM_TILES, NCH, CHUNK),
            nrm.reshape(NUM_TILES, NCH, CHUNK))


@jax.jit
def kernel(feature, edge_index, edge_index2, norm_A, norm_A_2,
           W1_1, b1_1, W1_2, b1_2, W2, b2, temp1, temp2):
    w1t = jnp.concatenate([W1_1, W1_2], axis=0).T
    b1 = jnp.concatenate([b1_1, b1_2])[None, :]

    blk = 1000
    x0 = pl.pallas_call(
        _mlp_body,
        grid=(N // blk,),
        in_specs=[
            pl.BlockSpec((blk, D_IN), lambda i: (i, 0)),
            pl.BlockSpec((D_IN, H), lambda i: (0, 0)),
            pl.BlockSpec((1, H), lambda i: (0, 0)),
        ],
        out_specs=pl.BlockSpec((NUM_CORES, blk, HH), lambda i: (0, i, 0)),
        out_shape=jax.ShapeDtypeStruct((NUM_CORES, N, HH), jnp.float32),
    )(feature, w1t, b1)

    s1, d1, n1 = _pad_edges(edge_index, norm_A)
    s2, d2, n2 = _pad_edges(edge_index2, norm_A_2)
    srcs = jnp.stack([s1, s2])
    dsts = jnp.stack([d1, d2])
    norms = jnp.stack([n1, n2])

    hops = _sc_prop(x0, srcs, dsts, norms)

    w2t = W2.T
    temps = jnp.stack([temp1.T, temp2.T])
    wsc = temps[:, :, :, None] * jnp.stack([w2t[:HH], w2t[HH:]])[:, None]

    out = pl.pallas_call(
        _proj_body,
        grid=(N // blk,),
        in_specs=[
            pl.BlockSpec((NUM_CORES, blk, HH), lambda i: (0, i, 0)),
            pl.BlockSpec((NUM_CORES, K, blk, HH), lambda i: (0, 0, i, 0)),
            pl.BlockSpec((NUM_CORES, K + 1, HH, N_CLASSES),
                         lambda i: (0, 0, 0, 0)),
            pl.BlockSpec((1, N_CLASSES), lambda i: (0, 0)),
        ],
        out_specs=pl.BlockSpec((blk, N_CLASSES), lambda i: (i, 0)),
        out_shape=jax.ShapeDtypeStruct((N, N_CLASSES), jnp.float32),
    )(x0, hops, wsc, b2[None, :])
    return out
